# baseline (speedup 1.0000x reference)
import functools
import math

import jax
import jax.numpy as jnp
from jax import lax
from jax.experimental import pallas as pl
from jax.experimental.pallas import tpu as pltpu
from jax.experimental.pallas import tpu_sc as plsc

N = 4096
S = 16
N_KINDS = 8
N_TYPES = 64
N_LAYERS = 4096
DM = 256
BF = 24
F2 = 2 * BF

NBA = 64
RA = N // NBA
SA = RA * S

NBB = 16
RBB = N // NBB

RC = 16
NCH = RA // RC

SC_NC = 2
SC_NS = 16
NW = SC_NC * SC_NS
BPW = N // NW


def _gelu_exact(x):
    return 0.5 * x * (1.0 + jax.lax.erf(x * (1.0 / math.sqrt(2.0))))


def _gelu_tanh(x):
    c = math.sqrt(2.0 / math.pi)
    return 0.5 * x * (1.0 + jnp.tanh(c * (x + 0.044715 * x * x * x)))


def _mm(a, b):
    return jax.lax.dot_general(a, b, (((1,), (0,)), ((), ())),
                               preferred_element_type=jnp.float32)


NF = 4
RF = (N * S // 8) // NF


def _fourier_kernel(v8_ref, E_ref, Bs_ref, P_ref, vf_ref):
    vrep = _mm(v8_ref[:], E_ref[:])
    t = vrep * Bs_ref[:] + P_ref[:]
    z = 2.0 * (t - jnp.round(t))
    w = z * z
    s = z * (3.1415442432 + w * (-5.1666561447 + w * (2.5437537138
        + w * (-0.5834068137 + w * 0.0647822111))))
    vf_ref[:] = s.astype(jnp.bfloat16)


def _expert_kernel(vals_ref, kf_ref, kfl_ref, maskl_ref, mask2_ref,
                   kemb_ref, W1_ref, b1_ref, W2_ref, b2_ref, hnum_ref):
    f32 = jnp.float32
    bf16 = jnp.bfloat16

    vfb = vals_ref[:]

    kf = kf_ref[:]
    onehot = (kf == jax.lax.broadcasted_iota(jnp.int32, (SA, N_KINDS), 1)).astype(f32)

    grp = jax.lax.broadcasted_iota(jnp.int32, (SA, N_KINDS * F2), 1) // F2
    xk = jnp.where(grp == kf, jnp.concatenate([vfb] * N_KINDS, axis=1),
                   jnp.bfloat16(0.0))
    h1 = _mm(xk, W1_ref[:])
    h1 = h1 + _mm(onehot, b1_ref[:])
    g = _gelu_exact(h1)

    kfl = kfl_ref[0]
    maskl = maskl_ref[0]
    rowsel = jax.lax.broadcasted_iota(jnp.int32, (N_KINDS * RC, 1), 0)
    lane_row = jax.lax.broadcasted_iota(jnp.int32, (1, RC * S), 1) // S
    gb = g.astype(bf16)
    ones_col = jnp.ones((RC * S, 1), bf16)
    G_chunks, hist_chunks = [], []
    for c in range(NCH):
        lo = c * RC * S
        cj = kfl[:, lo:lo + RC * S] * RC + lane_row
        Mw = ((rowsel == cj).astype(bf16)
              * maskl[:, lo:lo + RC * S].astype(bf16))
        G_chunks.append(_mm(Mw, gb[lo:lo + RC * S]))
        hist_chunks.append(_mm(Mw, ones_col))

    hsum = jnp.zeros((RA, DM), f32)
    for k in range(N_KINDS):
        G_k = jnp.concatenate([gc[k * RC:(k + 1) * RC] for gc in G_chunks],
                              axis=0)
        h_k = jnp.concatenate([hc[k * RC:(k + 1) * RC] for hc in hist_chunks],
                              axis=0)
        hsum = hsum + _mm(G_k.astype(bf16), W2_ref[k])
        hsum = hsum + h_k * (b2_ref[k] + kemb_ref[k])[None, :]

    m2 = mask2_ref[:]
    denom = jnp.clip(jnp.sum(m2, axis=1, keepdims=True), 1.0, None)
    hnum_ref[:] = hsum / denom


def _fusion_kernel(hnum_ref, trows_ref, lrows_ref, meta_ref,
                   fW1_ref, fb1_ref, fW2_ref, fb2_ref, mW_ref, mb_ref,
                   out_ref):
    bf16 = jnp.bfloat16
    fW1 = fW1_ref[:].astype(bf16)
    z = (_mm(hnum_ref[:].astype(bf16), fW1[0:DM])
         + _mm(trows_ref[:].astype(bf16), fW1[DM:2 * DM])
         + _mm(lrows_ref[:].astype(bf16), fW1[2 * DM:3 * DM]) + fb1_ref[:])
    z = _gelu_exact(z)
    fh = _mm(z.astype(bf16), fW2_ref[:].astype(bf16)) + fb2_ref[:]
    out_ref[:] = fh + _mm(meta_ref[:], mW_ref[:]) + mb_ref[:]


@functools.partial(
    pl.kernel,
    mesh=plsc.VectorSubcoreMesh(core_axis_name="c", subcore_axis_name="s"),
    out_type=[jax.ShapeDtypeStruct((N, DM), jnp.float32),
              jax.ShapeDtypeStruct((N, DM), jnp.float32)],
    scratch_types=[
        pltpu.VMEM((BPW,), jnp.int32),
        pltpu.VMEM((BPW, DM), jnp.float32),
        pltpu.SemaphoreType.DMA,
    ],
)
def _emb_gather(temb_hbm, lemb_hbm, pt_hbm, lid_hbm, trows_hbm, lrows_hbm,
                idx_v, rows_v, sem):
    wid = lax.axis_index("s") * SC_NC + lax.axis_index("c")
    base = wid * BPW
    pltpu.sync_copy(pt_hbm.at[pl.ds(base, BPW)], idx_v)
    pltpu.async_copy(temb_hbm.at[idx_v], rows_v, sem).wait()
    pltpu.sync_copy(rows_v, trows_hbm.at[pl.ds(base, BPW)])
    pltpu.sync_copy(lid_hbm.at[pl.ds(base, BPW)], idx_v)
    pltpu.async_copy(lemb_hbm.at[idx_v], rows_v, sem).wait()
    pltpu.sync_copy(rows_v, lrows_hbm.at[pl.ds(base, BPW)])


def kernel(values, kinds, mask, prim_type, layer_id, meta, B_mat, kind_emb,
           type_emb, layer_emb, W1s, b1s, W2s, b2s, fW1, fb1, fW2, fb2, mW, mb):
    f32 = jnp.float32
    v8 = values.reshape(N * S // 8, 8).astype(f32)
    kf = kinds.reshape(N * S, 1).astype(jnp.int32)
    kfl = kinds.reshape(NBA, 1, SA).astype(jnp.int32)
    maskl = mask.reshape(NBA, 1, SA).astype(f32)
    mask2 = mask.astype(f32)
    pt = prim_type.reshape(N).astype(jnp.int32)
    lid = layer_id.reshape(N).astype(jnp.int32)
    Bs = B_mat.reshape(1, BF).astype(f32)
    Bs2 = jnp.concatenate([Bs, Bs], axis=1)
    P2 = jnp.concatenate([jnp.zeros((1, BF), f32),
                          jnp.full((1, BF), 0.25, f32)], axis=1)
    E8 = jnp.repeat(jnp.eye(8, dtype=f32), F2, axis=1)
    Bs384 = jnp.tile(Bs2, (1, 8))
    P384 = jnp.tile(P2, (1, 8))

    full = lambda shape: pl.BlockSpec(shape, lambda i: tuple(0 for _ in shape))
    vf_flat = pl.pallas_call(
        _fourier_kernel,
        grid=(NF,),
        in_specs=[
            pl.BlockSpec((RF, 8), lambda i: (i, 0)),
            full((8, 8 * F2)),
            full((1, 8 * F2)),
            full((1, 8 * F2)),
        ],
        out_specs=pl.BlockSpec((RF, 8 * F2), lambda i: (i, 0)),
        out_shape=jax.ShapeDtypeStruct((N * S // 8, 8 * F2), jnp.bfloat16),
        compiler_params=pltpu.CompilerParams(
            dimension_semantics=("arbitrary",),
        ),
    )(v8, E8, Bs384, P384)
    vf = vf_flat.reshape(N * S, F2)
    h_num = pl.pallas_call(
        _expert_kernel,
        grid=(NBA,),
        in_specs=[
            pl.BlockSpec((SA, F2), lambda i: (i, 0)),
            pl.BlockSpec((SA, 1), lambda i: (i, 0)),
            pl.BlockSpec((1, 1, SA), lambda i: (i, 0, 0)),
            pl.BlockSpec((1, 1, SA), lambda i: (i, 0, 0)),
            pl.BlockSpec((RA, S), lambda i: (i, 0)),
            full((N_KINDS, DM)),
            full((N_KINDS * F2, DM)),
            full((N_KINDS, DM)),
            full((N_KINDS, DM, DM)),
            full((N_KINDS, DM)),
        ],
        out_specs=pl.BlockSpec((RA, DM), lambda i: (i, 0)),
        out_shape=jax.ShapeDtypeStruct((N, DM), f32),
        compiler_params=pltpu.CompilerParams(
            dimension_semantics=("arbitrary",),
        ),
    )(vf, kf, kfl, maskl, mask2, kind_emb,
      W1s.reshape(N_KINDS * F2, DM).astype(jnp.bfloat16),
      b1s, W2s.astype(jnp.bfloat16), b2s)
    t_rows, l_rows = _emb_gather(type_emb.astype(f32), layer_emb.astype(f32),
                                 pt, lid)
    out = pl.pallas_call(
        _fusion_kernel,
        grid=(NBB,),
        in_specs=[
            pl.BlockSpec((RBB, DM), lambda i: (i, 0)),
            pl.BlockSpec((RBB, DM), lambda i: (i, 0)),
            pl.BlockSpec((RBB, DM), lambda i: (i, 0)),
            pl.BlockSpec((RBB, 4), lambda i: (i, 0)),
            full((3 * DM, DM)),
            full((1, DM)),
            full((DM, DM)),
            full((1, DM)),
            full((4, DM)),
            full((1, DM)),
        ],
        out_specs=pl.BlockSpec((RBB, DM), lambda i: (i, 0)),
        out_shape=jax.ShapeDtypeStruct((N, DM), f32),
        compiler_params=pltpu.CompilerParams(
            dimension_semantics=("arbitrary",),
        ),
    )(h_num, t_rows, l_rows, meta.astype(f32),
      fW1, fb1.reshape(1, DM), fW2, fb2.reshape(1, DM), mW, mb.reshape(1, DM))
    return out


# speedup vs baseline: 1.1104x; 1.1104x over previous
import functools
import math

import jax
import jax.numpy as jnp
from jax import lax
from jax.experimental import pallas as pl
from jax.experimental.pallas import tpu as pltpu
from jax.experimental.pallas import tpu_sc as plsc

N = 4096
S = 16
N_KINDS = 8
N_TYPES = 64
N_LAYERS = 4096
DM = 256
BF = 24
F2 = 2 * BF

NBA = 16
RA = N // NBA
SA = RA * S

NBB = 16
RBB = N // NBB

RC = 16
NCH = RA // RC

SC_NC = 2
SC_NS = 16
NW = SC_NC * SC_NS
BPW = N // NW


def _gelu_exact(x):
    return 0.5 * x * (1.0 + jax.lax.erf(x * (1.0 / math.sqrt(2.0))))


def _gelu_tanh(x):
    c = math.sqrt(2.0 / math.pi)
    return 0.5 * x * (1.0 + jnp.tanh(c * (x + 0.044715 * x * x * x)))


def _mm(a, b):
    return jax.lax.dot_general(a, b, (((1,), (0,)), ((), ())),
                               preferred_element_type=jnp.float32)


NF = 4
RF = (N * S // 8) // NF


def _fourier_kernel(v8_ref, E_ref, Bs_ref, P_ref, vf_ref):
    vrep = _mm(v8_ref[:], E_ref[:])
    t = vrep * Bs_ref[:] + P_ref[:]
    z = 2.0 * (t - jnp.round(t))
    w = z * z
    s = z * (3.1415442432 + w * (-5.1666561447 + w * (2.5437537138
        + w * (-0.5834068137 + w * 0.0647822111))))
    vf_ref[:] = s.astype(jnp.bfloat16)


def _expert_kernel(vals_ref, kf_ref, kfl_ref, maskl_ref, mask2_ref,
                   kemb_ref, W1_ref, b1_ref, W2_ref, b2_ref, hnum_ref):
    f32 = jnp.float32
    bf16 = jnp.bfloat16

    vfb = vals_ref[:]

    kf = kf_ref[:]
    onehot = (kf == jax.lax.broadcasted_iota(jnp.int32, (SA, N_KINDS), 1)).astype(f32)

    grp = jax.lax.broadcasted_iota(jnp.int32, (SA, N_KINDS * F2), 1) // F2
    xk = jnp.where(grp == kf, jnp.concatenate([vfb] * N_KINDS, axis=1),
                   jnp.bfloat16(0.0))
    h1 = _mm(xk, W1_ref[:])
    h1 = h1 + _mm(onehot, b1_ref[:])
    g = _gelu_exact(h1)

    kfl = kfl_ref[0]
    maskl = maskl_ref[0]
    rowsel = jax.lax.broadcasted_iota(jnp.int32, (N_KINDS * RC, 1), 0)
    lane_row = jax.lax.broadcasted_iota(jnp.int32, (1, RC * S), 1) // S
    gb = g.astype(bf16)
    ones_col = jnp.ones((RC * S, 1), bf16)
    G_chunks, hist_chunks = [], []
    for c in range(NCH):
        lo = c * RC * S
        cj = kfl[:, lo:lo + RC * S] * RC + lane_row
        Mw = ((rowsel == cj).astype(bf16)
              * maskl[:, lo:lo + RC * S].astype(bf16))
        G_chunks.append(_mm(Mw, gb[lo:lo + RC * S]))
        hist_chunks.append(_mm(Mw, ones_col))

    hsum = jnp.zeros((RA, DM), f32)
    for k in range(N_KINDS):
        G_k = jnp.concatenate([gc[k * RC:(k + 1) * RC] for gc in G_chunks],
                              axis=0)
        h_k = jnp.concatenate([hc[k * RC:(k + 1) * RC] for hc in hist_chunks],
                              axis=0)
        hsum = hsum + _mm(G_k.astype(bf16), W2_ref[k])
        hsum = hsum + h_k * (b2_ref[k] + kemb_ref[k])[None, :]

    m2 = mask2_ref[:]
    denom = jnp.clip(jnp.sum(m2, axis=1, keepdims=True), 1.0, None)
    hnum_ref[:] = hsum / denom


def _fusion_kernel(hnum_ref, trows_ref, lrows_ref, meta_ref,
                   fW1_ref, fb1_ref, fW2_ref, fb2_ref, mW_ref, mb_ref,
                   out_ref):
    bf16 = jnp.bfloat16
    fW1 = fW1_ref[:].astype(bf16)
    z = (_mm(hnum_ref[:].astype(bf16), fW1[0:DM])
         + _mm(trows_ref[:].astype(bf16), fW1[DM:2 * DM])
         + _mm(lrows_ref[:].astype(bf16), fW1[2 * DM:3 * DM]) + fb1_ref[:])
    z = _gelu_exact(z)
    fh = _mm(z.astype(bf16), fW2_ref[:].astype(bf16)) + fb2_ref[:]
    out_ref[:] = fh + _mm(meta_ref[:], mW_ref[:]) + mb_ref[:]


@functools.partial(
    pl.kernel,
    mesh=plsc.VectorSubcoreMesh(core_axis_name="c", subcore_axis_name="s"),
    out_type=[jax.ShapeDtypeStruct((N, DM), jnp.float32),
              jax.ShapeDtypeStruct((N, DM), jnp.float32)],
    scratch_types=[
        pltpu.VMEM((BPW,), jnp.int32),
        pltpu.VMEM((BPW, DM), jnp.float32),
        pltpu.SemaphoreType.DMA,
    ],
)
def _emb_gather(temb_hbm, lemb_hbm, pt_hbm, lid_hbm, trows_hbm, lrows_hbm,
                idx_v, rows_v, sem):
    wid = lax.axis_index("s") * SC_NC + lax.axis_index("c")
    base = wid * BPW
    pltpu.sync_copy(pt_hbm.at[pl.ds(base, BPW)], idx_v)
    pltpu.async_copy(temb_hbm.at[idx_v], rows_v, sem).wait()
    pltpu.sync_copy(rows_v, trows_hbm.at[pl.ds(base, BPW)])
    pltpu.sync_copy(lid_hbm.at[pl.ds(base, BPW)], idx_v)
    pltpu.async_copy(lemb_hbm.at[idx_v], rows_v, sem).wait()
    pltpu.sync_copy(rows_v, lrows_hbm.at[pl.ds(base, BPW)])


def kernel(values, kinds, mask, prim_type, layer_id, meta, B_mat, kind_emb,
           type_emb, layer_emb, W1s, b1s, W2s, b2s, fW1, fb1, fW2, fb2, mW, mb):
    f32 = jnp.float32
    v8 = values.reshape(N * S // 8, 8).astype(f32)
    kf = kinds.reshape(N * S, 1).astype(jnp.int32)
    kfl = kinds.reshape(NBA, 1, SA).astype(jnp.int32)
    maskl = mask.reshape(NBA, 1, SA).astype(f32)
    mask2 = mask.astype(f32)
    pt = prim_type.reshape(N).astype(jnp.int32)
    lid = layer_id.reshape(N).astype(jnp.int32)
    Bs = B_mat.reshape(1, BF).astype(f32)
    Bs2 = jnp.concatenate([Bs, Bs], axis=1)
    P2 = jnp.concatenate([jnp.zeros((1, BF), f32),
                          jnp.full((1, BF), 0.25, f32)], axis=1)
    E8 = jnp.repeat(jnp.eye(8, dtype=f32), F2, axis=1)
    Bs384 = jnp.tile(Bs2, (1, 8))
    P384 = jnp.tile(P2, (1, 8))

    full = lambda shape: pl.BlockSpec(shape, lambda i: tuple(0 for _ in shape))
    vf_flat = pl.pallas_call(
        _fourier_kernel,
        grid=(NF,),
        in_specs=[
            pl.BlockSpec((RF, 8), lambda i: (i, 0)),
            full((8, 8 * F2)),
            full((1, 8 * F2)),
            full((1, 8 * F2)),
        ],
        out_specs=pl.BlockSpec((RF, 8 * F2), lambda i: (i, 0)),
        out_shape=jax.ShapeDtypeStruct((N * S // 8, 8 * F2), jnp.bfloat16),
        compiler_params=pltpu.CompilerParams(
            dimension_semantics=("arbitrary",),
        ),
    )(v8, E8, Bs384, P384)
    vf = vf_flat.reshape(N * S, F2)
    h_num = pl.pallas_call(
        _expert_kernel,
        grid=(NBA,),
        in_specs=[
            pl.BlockSpec((SA, F2), lambda i: (i, 0)),
            pl.BlockSpec((SA, 1), lambda i: (i, 0)),
            pl.BlockSpec((1, 1, SA), lambda i: (i, 0, 0)),
            pl.BlockSpec((1, 1, SA), lambda i: (i, 0, 0)),
            pl.BlockSpec((RA, S), lambda i: (i, 0)),
            full((N_KINDS, DM)),
            full((N_KINDS * F2, DM)),
            full((N_KINDS, DM)),
            full((N_KINDS, DM, DM)),
            full((N_KINDS, DM)),
        ],
        out_specs=pl.BlockSpec((RA, DM), lambda i: (i, 0)),
        out_shape=jax.ShapeDtypeStruct((N, DM), f32),
        compiler_params=pltpu.CompilerParams(
            dimension_semantics=("arbitrary",),
        ),
    )(vf, kf, kfl, maskl, mask2, kind_emb,
      W1s.reshape(N_KINDS * F2, DM).astype(jnp.bfloat16),
      b1s, W2s.astype(jnp.bfloat16), b2s)
    t_rows, l_rows = _emb_gather(type_emb.astype(f32), layer_emb.astype(f32),
                                 pt, lid)
    out = pl.pallas_call(
        _fusion_kernel,
        grid=(NBB,),
        in_specs=[
            pl.BlockSpec((RBB, DM), lambda i: (i, 0)),
            pl.BlockSpec((RBB, DM), lambda i: (i, 0)),
            pl.BlockSpec((RBB, DM), lambda i: (i, 0)),
            pl.BlockSpec((RBB, 4), lambda i: (i, 0)),
            full((3 * DM, DM)),
            full((1, DM)),
            full((DM, DM)),
            full((1, DM)),
            full((4, DM)),
            full((1, DM)),
        ],
        out_specs=pl.BlockSpec((RBB, DM), lambda i: (i, 0)),
        out_shape=jax.ShapeDtypeStruct((N, DM), f32),
        compiler_params=pltpu.CompilerParams(
            dimension_semantics=("arbitrary",),
        ),
    )(h_num, t_rows, l_rows, meta.astype(f32),
      fW1, fb1.reshape(1, DM), fW2, fb2.reshape(1, DM), mW, mb.reshape(1, DM))
    return out
